# Initial kernel scaffold; baseline (speedup 1.0000x reference)
import jax
import jax.numpy as jnp
from jax.experimental import pallas as pl


def kernel(x, edge_index, W1, b1, W2, b2):
    raise NotImplementedError("write your pallas kernel here")


# speedup vs baseline: 15.5944x; 15.5944x over previous
import functools

import jax
import jax.numpy as jnp
from jax import lax
from jax.experimental import pallas as pl
from jax.experimental.pallas import tpu as pltpu
from jax.experimental.pallas import tpu_sc as plsc

N = 10000
D = 256
E = 160000
SCALE = 1.8

NC = 2
NS = 16
L = 16

NPAD = 10240
EPAD = 163840
W = 128
PW = 128
ROWS_PER_TILE = NPAD // NS
EW_PER_TILE = EPAD // NS // W
EPT = EPAD // NS
NW = EPT // PW

_mesh = plsc.VectorSubcoreMesh(core_axis_name="c", subcore_axis_name="s")


@functools.partial(
    pl.kernel,
    out_type=jax.ShapeDtypeStruct((NPAD * 8,), jnp.float32),
    mesh=_mesh,
    scratch_types=[
        pltpu.VMEM((W,), jnp.int32),
        pltpu.VMEM((W,), jnp.int32),
        pltpu.VMEM((W,), jnp.float32),
        pltpu.VMEM((ROWS_PER_TILE * 8,), jnp.float32),
        pltpu.VMEM_SHARED((NPAD * 8,), jnp.float32),
    ],
)
def _sc_degree(col_hbm, deg8_hbm, colv, colv2, ones, zbuf, deg_sh):
    cid = lax.axis_index("c")
    sid = lax.axis_index("s")
    zeros16 = jnp.zeros((L,), jnp.float32)
    for i in range(ROWS_PER_TILE * 8 // L):
        zbuf[pl.ds(i * L, L)] = zeros16
    for i in range(W // L):
        ones[pl.ds(i * L, L)] = zeros16 + 1.0
    pltpu.sync_copy(
        zbuf, deg_sh.at[pl.ds(sid * ROWS_PER_TILE * 8, ROWS_PER_TILE * 8)])
    plsc.subcore_barrier()

    lane8 = lax.broadcasted_iota(jnp.int32, (L,), 0) & 7

    def body(w, _):
        base = sid * (EPAD // NS) + w * W
        pltpu.sync_copy(col_hbm.at[pl.ds(base, W)], colv)
        for j in range(W // L):
            colv2[pl.ds(j * L, L)] = colv[pl.ds(j * L, L)] * 8 + lane8
        pltpu.sync_copy(ones, deg_sh.at[colv2], add=True)
        return _

    lax.fori_loop(0, EW_PER_TILE, body, None)
    plsc.subcore_barrier()

    half = ROWS_PER_TILE * 8 // 2
    off = cid * (NPAD * 8 // 2) + sid * half
    pltpu.sync_copy(deg_sh.at[pl.ds(off, half)], zbuf.at[pl.ds(0, half)])
    pltpu.sync_copy(zbuf.at[pl.ds(0, half)], deg8_hbm.at[pl.ds(off, half)])


def _tc_prep_body(x_ref, w1_ref, b1_ref, w2_ref, b2_ref, deg_ref, y_ref):
    i = pl.program_id(0)
    x = x_ref[...]
    dn = (((1,), (1,)), ((), ()))
    h1 = lax.dot_general(x, w1_ref[...], dn,
                         preferred_element_type=jnp.float32,
                         precision=lax.Precision.HIGHEST) + b1_ref[...]
    h2 = lax.dot_general(x, w2_ref[...], dn,
                         preferred_element_type=jnp.float32,
                         precision=lax.Precision.HIGHEST) + b2_ref[...]
    nrm = jnp.sqrt(jnp.sum(h2 * h2, axis=1, keepdims=True))
    h2 = h2 / jnp.maximum(nrm, 1e-12) * SCALE
    deg = jnp.sum(deg_ref[...], axis=1, keepdims=True) + 1.0
    dis = lax.rsqrt(deg)
    rows = i * 256 + lax.broadcasted_iota(jnp.int32, (256, 1), 0)
    s = dis * (rows < N).astype(jnp.float32)
    y1 = h1 * s
    y2 = h2 * s
    y_ref[0] = y1[:, :128]
    y_ref[1] = y1[:, 128:]
    y_ref[2] = y2[:, :128]
    y_ref[3] = y2[:, 128:]


_tc_prep = pl.pallas_call(
    _tc_prep_body,
    grid=(NPAD // 256,),
    in_specs=[
        pl.BlockSpec((256, D), lambda i: (i, 0)),
        pl.BlockSpec((D, D), lambda i: (0, 0)),
        pl.BlockSpec((1, D), lambda i: (0, 0)),
        pl.BlockSpec((D, D), lambda i: (0, 0)),
        pl.BlockSpec((1, D), lambda i: (0, 0)),
        pl.BlockSpec((256, 8), lambda i: (i, 0)),
    ],
    out_specs=pl.BlockSpec((4, 256, 128), lambda i: (0, i, 0)),
    out_shape=jax.ShapeDtypeStruct((4, NPAD, 128), jnp.float32),
)


@functools.partial(
    pl.kernel,
    out_type=jax.ShapeDtypeStruct((4 * NPAD, 128), jnp.float32),
    mesh=_mesh,
    scratch_types=[
        pltpu.VMEM((EPT,), jnp.int32),
        pltpu.VMEM((PW,), jnp.int32),
        pltpu.VMEM((PW,), jnp.int32),
        pltpu.VMEM((PW, 128), jnp.float32),
        pltpu.VMEM((PW, 128), jnp.float32),
        pltpu.VMEM_SHARED((NPAD, 128), jnp.float32),
        pltpu.SemaphoreType.DMA,
        pltpu.SemaphoreType.DMA,
        pltpu.SemaphoreType.DMA,
        pltpu.SemaphoreType.DMA,
    ],
)
def _sc_prop(row_hbm, col2_hbm, y_hbm, acc_hbm,
             rowv2, colw0, colw1, g0, g1,
             acc_sh, sem0, sem1, semc0, semc1):
    cid = lax.axis_index("c")
    sid = lax.axis_index("s")

    def gslice(w):
        return y_hbm.at[rowv2.at[pl.ds(pl.multiple_of(w * PW, PW), PW)]]

    def cslice(w):
        return col2_hbm.at[sid * NW + w]

    for q in range(2):
        qq = cid * 2 + q
        yoff = qq * NPAD
        pltpu.sync_copy(row_hbm.at[pl.ds(sid * EPT, EPT)], rowv2)

        def addoff(i, _):
            off = pl.multiple_of(i * L, L)
            rowv2[pl.ds(off, L)] = rowv2[pl.ds(off, L)] + yoff
            return _

        lax.fori_loop(0, EPT // L, addoff, None)
        pltpu.sync_copy(
            y_hbm.at[pl.ds(yoff + sid * ROWS_PER_TILE, ROWS_PER_TILE), :],
            acc_sh.at[pl.ds(sid * ROWS_PER_TILE, ROWS_PER_TILE), :])
        plsc.subcore_barrier()

        pltpu.async_copy(cslice(0), colw0, semc0)
        pltpu.async_copy(gslice(0), g0, sem0)

        def body(k, _):
            w = k * 2
            pltpu.async_copy(gslice(w + 1), g1, sem1)
            pltpu.async_copy(cslice(w + 1), colw1, semc1)
            pltpu.make_async_copy(gslice(w), g0, sem0).wait()
            pltpu.make_async_copy(cslice(w), colw0, semc0).wait()
            pltpu.sync_copy(g0, acc_sh.at[colw0], add=True)
            pltpu.async_copy(gslice(w + 2), g0, sem0)
            pltpu.async_copy(cslice(w + 2), colw0, semc0)
            pltpu.make_async_copy(gslice(w + 1), g1, sem1).wait()
            pltpu.make_async_copy(cslice(w + 1), colw1, semc1).wait()
            pltpu.sync_copy(g1, acc_sh.at[colw1], add=True)
            return _

        lax.fori_loop(0, NW // 2 - 1, body, None)
        w = NW - 2
        pltpu.async_copy(gslice(w + 1), g1, sem1)
        pltpu.async_copy(cslice(w + 1), colw1, semc1)
        pltpu.make_async_copy(gslice(w), g0, sem0).wait()
        pltpu.make_async_copy(cslice(w), colw0, semc0).wait()
        pltpu.sync_copy(g0, acc_sh.at[colw0], add=True)
        pltpu.make_async_copy(gslice(w + 1), g1, sem1).wait()
        pltpu.make_async_copy(cslice(w + 1), colw1, semc1).wait()
        pltpu.sync_copy(g1, acc_sh.at[colw1], add=True)
        plsc.subcore_barrier()
        pltpu.sync_copy(
            acc_sh.at[pl.ds(sid * ROWS_PER_TILE, ROWS_PER_TILE), :],
            acc_hbm.at[pl.ds(yoff + sid * ROWS_PER_TILE, ROWS_PER_TILE), :])
        plsc.subcore_barrier()


def _tc_final_body(acc_ref, deg_ref, z_ref, mu_ref):
    deg = jnp.sum(deg_ref[...], axis=1, keepdims=True) + 1.0
    dis = lax.rsqrt(deg)
    mu_ref[...] = jnp.concatenate([acc_ref[0], acc_ref[1]], axis=1) * dis
    z_ref[...] = jnp.concatenate([acc_ref[2], acc_ref[3]], axis=1) * dis


_tc_final = pl.pallas_call(
    _tc_final_body,
    grid=(NPAD // 256,),
    in_specs=[
        pl.BlockSpec((4, 256, 128), lambda i: (0, i, 0)),
        pl.BlockSpec((256, 8), lambda i: (i, 0)),
    ],
    out_specs=[
        pl.BlockSpec((256, D), lambda i: (i, 0)),
        pl.BlockSpec((256, D), lambda i: (i, 0)),
    ],
    out_shape=[
        jax.ShapeDtypeStruct((NPAD, D), jnp.float32),
        jax.ShapeDtypeStruct((NPAD, D), jnp.float32),
    ],
)


def kernel(x, edge_index, W1, b1, W2, b2):
    row = edge_index[0].astype(jnp.int32)
    col = edge_index[1].astype(jnp.int32)
    pad = N + (jnp.arange(EPAD - E, dtype=jnp.int32) % (NPAD - N))
    row_p = jnp.concatenate([row, pad])
    col_p = jnp.concatenate([col, pad])
    x_p = jnp.pad(x, ((0, NPAD - N), (0, 0)))

    deg8 = _sc_degree(col_p).reshape(NPAD, 8)
    y4 = _tc_prep(x_p, W1, b1.reshape(1, D), W2, b2.reshape(1, D), deg8)
    acc = _sc_prop(row_p, col_p.reshape(EPAD // PW, PW),
                   y4.reshape(4 * NPAD, 128))
    z, mu = _tc_final(acc.reshape(4, NPAD, 128), deg8)
    return (z[:N], mu[:N])


# speedup vs baseline: 16.1254x; 1.0341x over previous
import functools

import jax
import jax.numpy as jnp
from jax import lax
from jax.experimental import pallas as pl
from jax.experimental.pallas import tpu as pltpu
from jax.experimental.pallas import tpu_sc as plsc

N = 10000
D = 256
E = 160000
SCALE = 1.8

NC = 2
NS = 16
L = 16

NPAD = 10240
EPAD = 163840
W = 128
PW = 128
ROWS_PER_TILE = NPAD // NS
EW_PER_TILE = EPAD // NS // W
EPT = EPAD // NS
NW = EPT // PW

_mesh = plsc.VectorSubcoreMesh(core_axis_name="c", subcore_axis_name="s")


@functools.partial(
    pl.kernel,
    out_type=jax.ShapeDtypeStruct((NC, NPAD * 8), jnp.float32),
    mesh=_mesh,
    scratch_types=[
        pltpu.VMEM((W,), jnp.int32),
        pltpu.VMEM((W,), jnp.int32),
        pltpu.VMEM((W,), jnp.float32),
        pltpu.VMEM((ROWS_PER_TILE * 8,), jnp.float32),
        pltpu.VMEM_SHARED((NPAD * 8,), jnp.float32),
    ],
)
def _sc_degree(col_hbm, deg8_hbm, colv, colv2, ones, zbuf, deg_sh):
    cid = lax.axis_index("c")
    sid = lax.axis_index("s")
    zeros16 = jnp.zeros((L,), jnp.float32)
    for i in range(ROWS_PER_TILE * 8 // L):
        zbuf[pl.ds(i * L, L)] = zeros16
    for i in range(W // L):
        ones[pl.ds(i * L, L)] = zeros16 + 1.0
    pltpu.sync_copy(
        zbuf, deg_sh.at[pl.ds(sid * ROWS_PER_TILE * 8, ROWS_PER_TILE * 8)])
    plsc.subcore_barrier()

    lane8 = lax.broadcasted_iota(jnp.int32, (L,), 0) & 7

    def body(w, _):
        base = cid * (EPAD // NC) + sid * (EPAD // NC // NS) + w * W
        pltpu.sync_copy(col_hbm.at[pl.ds(base, W)], colv)
        for j in range(W // L):
            colv2[pl.ds(j * L, L)] = colv[pl.ds(j * L, L)] * 8 + lane8
        pltpu.sync_copy(ones, deg_sh.at[colv2], add=True)
        return _

    lax.fori_loop(0, EW_PER_TILE // NC, body, None)
    plsc.subcore_barrier()

    sl = ROWS_PER_TILE * 8
    pltpu.sync_copy(deg_sh.at[pl.ds(sid * sl, sl)], zbuf)
    pltpu.sync_copy(zbuf, deg8_hbm.at[cid, pl.ds(sid * sl, sl)])


def _tc_prep_body(x_ref, w1_ref, b1_ref, w2_ref, b2_ref, deg_ref, y_ref):
    i = pl.program_id(0)
    x = x_ref[...]
    dn = (((1,), (1,)), ((), ()))
    h1 = lax.dot_general(x, w1_ref[...], dn,
                         preferred_element_type=jnp.float32,
                         precision=lax.Precision.HIGHEST) + b1_ref[...]
    h2 = lax.dot_general(x, w2_ref[...], dn,
                         preferred_element_type=jnp.float32,
                         precision=lax.Precision.HIGHEST) + b2_ref[...]
    nrm = jnp.sqrt(jnp.sum(h2 * h2, axis=1, keepdims=True))
    h2 = h2 / jnp.maximum(nrm, 1e-12) * SCALE
    deg = jnp.sum(deg_ref[...], axis=(0, 2)).reshape(256, 1) + 1.0
    dis = lax.rsqrt(deg)
    rows = i * 256 + lax.broadcasted_iota(jnp.int32, (256, 1), 0)
    s = dis * (rows < N).astype(jnp.float32)
    y1 = h1 * s
    y2 = h2 * s
    y_ref[0] = y1[:, :128]
    y_ref[1] = y1[:, 128:]
    y_ref[2] = y2[:, :128]
    y_ref[3] = y2[:, 128:]


_tc_prep = pl.pallas_call(
    _tc_prep_body,
    grid=(NPAD // 256,),
    in_specs=[
        pl.BlockSpec((256, D), lambda i: (i, 0)),
        pl.BlockSpec((D, D), lambda i: (0, 0)),
        pl.BlockSpec((1, D), lambda i: (0, 0)),
        pl.BlockSpec((D, D), lambda i: (0, 0)),
        pl.BlockSpec((1, D), lambda i: (0, 0)),
        pl.BlockSpec((NC, 256, 8), lambda i: (0, i, 0)),
    ],
    out_specs=pl.BlockSpec((4, 256, 128), lambda i: (0, i, 0)),
    out_shape=jax.ShapeDtypeStruct((4, NPAD, 128), jnp.float32),
)


@functools.partial(
    pl.kernel,
    out_type=jax.ShapeDtypeStruct((4 * NPAD, 128), jnp.float32),
    mesh=_mesh,
    scratch_types=[
        pltpu.VMEM((EPT,), jnp.int32),
        pltpu.VMEM((PW,), jnp.int32),
        pltpu.VMEM((PW,), jnp.int32),
        pltpu.VMEM((PW, 128), jnp.float32),
        pltpu.VMEM((PW, 128), jnp.float32),
        pltpu.VMEM_SHARED((NPAD, 128), jnp.float32),
        pltpu.SemaphoreType.DMA,
        pltpu.SemaphoreType.DMA,
        pltpu.SemaphoreType.DMA,
        pltpu.SemaphoreType.DMA,
    ],
)
def _sc_prop(row_hbm, col2_hbm, y_hbm, acc_hbm,
             rowv2, colw0, colw1, g0, g1,
             acc_sh, sem0, sem1, semc0, semc1):
    cid = lax.axis_index("c")
    sid = lax.axis_index("s")

    def gslice(w):
        return y_hbm.at[rowv2.at[pl.ds(pl.multiple_of(w * PW, PW), PW)]]

    def cslice(w):
        return col2_hbm.at[sid * NW + w]

    for q in range(2):
        qq = cid * 2 + q
        yoff = qq * NPAD
        pltpu.sync_copy(row_hbm.at[pl.ds(sid * EPT, EPT)], rowv2)

        def addoff(i, _):
            off = pl.multiple_of(i * L, L)
            rowv2[pl.ds(off, L)] = rowv2[pl.ds(off, L)] + yoff
            return _

        lax.fori_loop(0, EPT // L, addoff, None)
        pltpu.sync_copy(
            y_hbm.at[pl.ds(yoff + sid * ROWS_PER_TILE, ROWS_PER_TILE), :],
            acc_sh.at[pl.ds(sid * ROWS_PER_TILE, ROWS_PER_TILE), :])
        plsc.subcore_barrier()

        pltpu.async_copy(cslice(0), colw0, semc0)
        pltpu.async_copy(gslice(0), g0, sem0)

        def body(k, _):
            w = k * 2
            pltpu.async_copy(gslice(w + 1), g1, sem1)
            pltpu.async_copy(cslice(w + 1), colw1, semc1)
            pltpu.make_async_copy(gslice(w), g0, sem0).wait()
            pltpu.make_async_copy(cslice(w), colw0, semc0).wait()
            pltpu.sync_copy(g0, acc_sh.at[colw0], add=True)
            pltpu.async_copy(gslice(w + 2), g0, sem0)
            pltpu.async_copy(cslice(w + 2), colw0, semc0)
            pltpu.make_async_copy(gslice(w + 1), g1, sem1).wait()
            pltpu.make_async_copy(cslice(w + 1), colw1, semc1).wait()
            pltpu.sync_copy(g1, acc_sh.at[colw1], add=True)
            return _

        lax.fori_loop(0, NW // 2 - 1, body, None)
        w = NW - 2
        pltpu.async_copy(gslice(w + 1), g1, sem1)
        pltpu.async_copy(cslice(w + 1), colw1, semc1)
        pltpu.make_async_copy(gslice(w), g0, sem0).wait()
        pltpu.make_async_copy(cslice(w), colw0, semc0).wait()
        pltpu.sync_copy(g0, acc_sh.at[colw0], add=True)
        pltpu.make_async_copy(gslice(w + 1), g1, sem1).wait()
        pltpu.make_async_copy(cslice(w + 1), colw1, semc1).wait()
        pltpu.sync_copy(g1, acc_sh.at[colw1], add=True)
        plsc.subcore_barrier()
        pltpu.sync_copy(
            acc_sh.at[pl.ds(sid * ROWS_PER_TILE, ROWS_PER_TILE), :],
            acc_hbm.at[pl.ds(yoff + sid * ROWS_PER_TILE, ROWS_PER_TILE), :])
        plsc.subcore_barrier()


def _tc_final_body(acc_ref, deg_ref, z_ref, mu_ref):
    deg = jnp.sum(deg_ref[...], axis=(0, 2)).reshape(256, 1) + 1.0
    dis = lax.rsqrt(deg)
    mu_ref[...] = jnp.concatenate([acc_ref[0], acc_ref[1]], axis=1) * dis
    z_ref[...] = jnp.concatenate([acc_ref[2], acc_ref[3]], axis=1) * dis


_tc_final = pl.pallas_call(
    _tc_final_body,
    grid=(NPAD // 256,),
    in_specs=[
        pl.BlockSpec((4, 256, 128), lambda i: (0, i, 0)),
        pl.BlockSpec((NC, 256, 8), lambda i: (0, i, 0)),
    ],
    out_specs=[
        pl.BlockSpec((256, D), lambda i: (i, 0)),
        pl.BlockSpec((256, D), lambda i: (i, 0)),
    ],
    out_shape=[
        jax.ShapeDtypeStruct((NPAD, D), jnp.float32),
        jax.ShapeDtypeStruct((NPAD, D), jnp.float32),
    ],
)


def kernel(x, edge_index, W1, b1, W2, b2):
    row = edge_index[0].astype(jnp.int32)
    col = edge_index[1].astype(jnp.int32)
    pad = N + (jnp.arange(EPAD - E, dtype=jnp.int32) % (NPAD - N))
    row_p = jnp.concatenate([row, pad])
    col_p = jnp.concatenate([col, pad])
    x_p = jnp.pad(x, ((0, NPAD - N), (0, 0)))

    deg8 = _sc_degree(col_p).reshape(NC, NPAD, 8)
    y4 = _tc_prep(x_p, W1, b1.reshape(1, D), W2, b2.reshape(1, D), deg8)
    acc = _sc_prop(row_p, col_p.reshape(EPAD // PW, PW),
                   y4.reshape(4 * NPAD, 128))
    z, mu = _tc_final(acc.reshape(4, NPAD, 128), deg8)
    return (z[:N], mu[:N])
